# baseline (speedup 1.0000x reference)
import jax
import jax.numpy as jnp
from jax import lax
from jax.experimental import pallas as pl
from jax.experimental.pallas import tpu as pltpu
from jax.experimental.pallas import tpu_sc as plsc

D = 64
NC, NS, L = 2, 16, 16
NW = NC * NS

B_TOTAL = 4096 * 200
B_PER_W = B_TOTAL // NW
CHUNK = 512
N_CHUNKS = B_PER_W // CHUNK
GRP = CHUNK // L


def _emb_body(table_hbm, idx_hbm, out_hbm, idx_v, rows_v, red_v, sem):
    wid = lax.axis_index("s") * NC + lax.axis_index("c")
    base0 = wid * B_PER_W

    def chunk_body(g, carry):
        base = base0 + g * CHUNK
        pltpu.sync_copy(idx_hbm.at[pl.ds(base, CHUNK)], idx_v)
        pltpu.async_copy(table_hbm.at[idx_v], rows_v, sem).wait()

        def mred(j, acc):
            return acc | (idx_v[pl.ds(j * L, L)] == 0)

        mv = lax.fori_loop(1, GRP, mred, idx_v[pl.ds(0, L)] == 0)
        nz = plsc.all_reduce_population_count(mv)[0]

        @pl.when(nz > 0)
        def _fixup():
            zeros = jnp.zeros((L,), jnp.float32)

            def fix_group(j, c2):
                v = idx_v[pl.ds(j * L, L)]
                m = v == 0
                rowids = j * L + lax.iota(jnp.int32, L)

                def fk(k, c3):
                    colids = lax.broadcast(k, (L,))
                    plsc.store_scatter(rows_v, [rowids, colids], zeros,
                                       mask=m)
                    return c3

                lax.fori_loop(0, D, fk, 0)
                return c2

            lax.fori_loop(0, GRP, fix_group, 0)

        pltpu.sync_copy(rows_v, out_hbm.at[pl.ds(base, CHUNK)])
        return carry

    lax.fori_loop(0, N_CHUNKS, chunk_body, 0)


@jax.jit
def _emb(idx_flat, table):
    mesh = plsc.VectorSubcoreMesh(core_axis_name="c", subcore_axis_name="s")
    f = pl.kernel(
        _emb_body,
        out_type=jax.ShapeDtypeStruct((B_TOTAL, D), jnp.float32),
        mesh=mesh,
        compiler_params=pltpu.CompilerParams(needs_layout_passes=False,
                                             use_tc_tiling_on_sc=False),
        scratch_types=[
            pltpu.VMEM((CHUNK,), jnp.int32),
            pltpu.VMEM((CHUNK, D), jnp.float32),
            pltpu.VMEM((L,), jnp.int32),
            pltpu.SemaphoreType.DMA,
        ],
    )
    return f(table, idx_flat)


def kernel(inputs, shared_weights):
    idx = inputs.reshape(-1).astype(jnp.int32)
    out = _emb(idx, shared_weights)
    return out.reshape(inputs.shape + (D,))


# speedup vs baseline: 1.0521x; 1.0521x over previous
import jax
import jax.numpy as jnp
from jax import lax
from jax.experimental import pallas as pl
from jax.experimental.pallas import tpu as pltpu
from jax.experimental.pallas import tpu_sc as plsc

D = 64
NC, NS, L = 2, 16, 16
NW = NC * NS

B_TOTAL = 4096 * 200
B_PER_W = B_TOTAL // NW
CHUNK = 512
N_CHUNKS = B_PER_W // CHUNK
GRP = CHUNK // L


def _scan_and_fixup(idx_v, rows_v):
    def mred(j, acc):
        return acc | (idx_v[pl.ds(j * L, L)] == 0)

    mv = lax.fori_loop(1, GRP, mred, idx_v[pl.ds(0, L)] == 0)
    nz = plsc.all_reduce_population_count(mv)[0]

    @pl.when(nz > 0)
    def _fixup():
        zeros = jnp.zeros((L,), jnp.float32)

        def fix_group(j, c2):
            v = idx_v[pl.ds(j * L, L)]
            m = v == 0
            rowids = j * L + lax.iota(jnp.int32, L)

            def fk(k, c3):
                colids = lax.broadcast(k, (L,))
                plsc.store_scatter(rows_v, [rowids, colids], zeros, mask=m)
                return c3

            lax.fori_loop(0, D, fk, 0)
            return c2

        lax.fori_loop(0, GRP, fix_group, 0)


def _emb_body(table_hbm, idx_hbm, out_hbm,
              idx0, idx1, rows0, rows1,
              isem0, isem1, gsem0, gsem1, wsem0, wsem1):
    wid = lax.axis_index("s") * NC + lax.axis_index("c")
    base0 = wid * B_PER_W

    idx_bufs = (idx0, idx1)
    rows_bufs = (rows0, rows1)
    isems = (isem0, isem1)
    gsems = (gsem0, gsem1)
    wsems = (wsem0, wsem1)

    def idx_src(g):
        return idx_hbm.at[pl.ds(base0 + g * CHUNK, CHUNK)]

    def out_dst(g):
        return out_hbm.at[pl.ds(base0 + g * CHUNK, CHUNK)]

    pltpu.async_copy(idx_src(0), idx0, isem0)
    pltpu.async_copy(idx_src(1), idx1, isem1)
    pltpu.make_async_copy(idx_src(0), idx0, isem0).wait()
    pltpu.async_copy(table_hbm.at[idx0], rows0, gsem0)

    def phase(g, p):
        q = 1 - p
        idx_p, rows_p = idx_bufs[p], rows_bufs[p]
        idx_q, rows_q = idx_bufs[q], rows_bufs[q]

        pltpu.make_async_copy(table_hbm.at[idx_p], rows_p, gsems[p]).wait()

        _scan_and_fixup(idx_p, rows_p)

        @pl.when(g + 2 < N_CHUNKS)
        def _pref():
            pltpu.async_copy(idx_src(g + 2), idx_p, isems[p])

        @pl.when(g + 1 < N_CHUNKS)
        def _next():
            pltpu.make_async_copy(idx_src(g + 1), idx_q, isems[q]).wait()

            @pl.when(g >= 1)
            def _drain():
                pltpu.make_async_copy(rows_q, out_dst(g - 1),
                                      wsems[q]).wait()

            pltpu.async_copy(table_hbm.at[idx_q], rows_q, gsems[q])

        pltpu.async_copy(rows_p, out_dst(g), wsems[p])

    def body(i, carry):
        g = i * 2
        phase(g, 0)
        phase(g + 1, 1)
        return carry

    lax.fori_loop(0, N_CHUNKS // 2, body, 0)

    pltpu.make_async_copy(rows0, out_dst(N_CHUNKS - 2), wsems[0]).wait()
    pltpu.make_async_copy(rows1, out_dst(N_CHUNKS - 1), wsems[1]).wait()


@jax.jit
def _emb(idx_flat, table):
    mesh = plsc.VectorSubcoreMesh(core_axis_name="c", subcore_axis_name="s")
    f = pl.kernel(
        _emb_body,
        out_type=jax.ShapeDtypeStruct((B_TOTAL, D), jnp.float32),
        mesh=mesh,
        compiler_params=pltpu.CompilerParams(needs_layout_passes=False,
                                             use_tc_tiling_on_sc=False),
        scratch_types=[
            pltpu.VMEM((CHUNK,), jnp.int32),
            pltpu.VMEM((CHUNK,), jnp.int32),
            pltpu.VMEM((CHUNK, D), jnp.float32),
            pltpu.VMEM((CHUNK, D), jnp.float32),
            pltpu.SemaphoreType.DMA,
            pltpu.SemaphoreType.DMA,
            pltpu.SemaphoreType.DMA,
            pltpu.SemaphoreType.DMA,
            pltpu.SemaphoreType.DMA,
            pltpu.SemaphoreType.DMA,
        ],
    )
    return f(table, idx_flat)


def kernel(inputs, shared_weights):
    idx = inputs.reshape(-1).astype(jnp.int32)
    out = _emb(idx, shared_weights)
    return out.reshape(inputs.shape + (D,))
